# Initial kernel scaffold; baseline (speedup 1.0000x reference)
import jax
import jax.numpy as jnp
from jax.experimental import pallas as pl


def kernel(x, batch, W_in, b_in, W_agg, b_agg, W_out, b_out):
    raise NotImplementedError("write your pallas kernel here")


# speedup vs baseline: 18.6050x; 18.6050x over previous
import functools

import jax
import jax.numpy as jnp
from jax import lax
from jax.experimental import pallas as pl
from jax.experimental.pallas import tpu as pltpu

N = 50000
IN_DIM = 128
OUT_DIM = 64
FEAT_DIM = 128
N_AGG = 8
S = 1024
B = 400
NB = N // B


def _pass_a(x_ref, brow_ref, bcol_ref, Win_ref, bin_ref, Wagg_ref, bagg_ref,
            Wm_ref, Wd_ref,
            dz_ref, sums_ref, maxtab_ref,
            acc_sums, acc_max, edges_scr):
    i = pl.program_id(0)

    @pl.when(i == 0)
    def _init():
        acc_sums[...] = jnp.zeros_like(acc_sums)
        acc_max[...] = jnp.full_like(acc_max, -jnp.inf)

    x = x_ref[...]
    xp = x @ Win_ref[...] + bin_ref[...]
    sc = jnp.exp(-jnp.abs(xp @ Wagg_ref[...] + bagg_ref[...]))
    d = x @ Wd_ref[0:IN_DIM, :] + xp @ Wd_ref[IN_DIM:IN_DIM + FEAT_DIM, :]
    y = xp @ Wm_ref[...]
    z = sc[:, 0:1] * y[:, 0:OUT_DIM]
    for a in range(1, N_AGG):
        z = z + sc[:, a:a + 1] * y[:, a * OUT_DIM:(a + 1) * OUT_DIM]
    dz_ref[...] = jnp.concatenate([d, z], axis=1)

    brow = brow_ref[0]
    oh = (lax.broadcasted_iota(jnp.int32, (S, B), 0) == brow).astype(jnp.float32)
    z_aug = jnp.concatenate([z, jnp.ones((B, OUT_DIM), jnp.float32)], axis=1)
    acc_sums[...] += jax.lax.dot_general(
        oh, z_aug, (((1,), (0,)), ((), ())), preferred_element_type=jnp.float32)

    for a in range(N_AGG):
        edges_scr[:, a * FEAT_DIM:(a + 1) * FEAT_DIM] = sc[:, a:a + 1] * xp
    bcol = bcol_ref[0]
    s_lo = brow_ref[0, 0, 0]
    s_hi = brow_ref[0, 0, B - 1]

    def body(s, _):
        m = bcol == s
        red = jnp.max(jnp.where(m, edges_scr[...], -jnp.inf), axis=0, keepdims=True)
        acc_max[pl.ds(s, 1), :] = jnp.maximum(acc_max[pl.ds(s, 1), :], red)
        return 0

    lax.fori_loop(s_lo, s_hi + 1, body, 0)

    @pl.when(i == NB - 1)
    def _fin():
        sums_ref[...] = acc_sums[...]
        maxtab_ref[...] = acc_max[...]


def _pass_b(dz_ref, bcol_ref, sums_ref, max_ref, Wx_ref, bout_ref,
            out_ref, T_scr):
    i = pl.program_id(0)

    @pl.when(i == 0)
    def _build_table():
        counts = sums_ref[:, OUT_DIM:OUT_DIM + 1]
        mean_part = sums_ref[:, 0:OUT_DIM] / jnp.maximum(counts, 1.0)
        mm = jnp.where(counts > 0, max_ref[...], 0.0)
        T_scr[...] = mean_part + jax.lax.dot_general(
            mm, Wx_ref[...], (((1,), (0,)), ((), ())),
            preferred_element_type=jnp.float32)

    bcol = bcol_ref[0]
    oh = (lax.broadcasted_iota(jnp.int32, (B, S), 1) == bcol).astype(jnp.float32)
    g = jax.lax.dot_general(oh, T_scr[...], (((1,), (0,)), ((), ())),
                            preferred_element_type=jnp.float32)
    out_ref[...] = jnp.maximum(dz_ref[:, 0:OUT_DIM] + g + bout_ref[...], 0.0)


@jax.jit
def kernel(x, batch, W_in, b_in, W_agg, b_agg, W_out, b_out):
    batch = batch.astype(jnp.int32)
    brow = batch.reshape(NB, 1, B)
    bcol = batch.reshape(NB, B, 1)
    Wd = W_out[0:IN_DIM + FEAT_DIM, :]
    Wtail = W_out[IN_DIM + FEAT_DIM:, :].reshape(N_AGG, 2 * FEAT_DIM, OUT_DIM)
    Wm = Wtail[:, 0:FEAT_DIM, :]
    Wx = Wtail[:, FEAT_DIM:, :]
    Wm_cat = jnp.transpose(Wm, (1, 0, 2)).reshape(FEAT_DIM, N_AGG * OUT_DIM)
    Wx_cat = Wx.reshape(N_AGG * FEAT_DIM, OUT_DIM)
    WaggP = jnp.zeros((FEAT_DIM, 128), jnp.float32).at[:, 0:N_AGG].set(W_agg)
    baggP = jnp.zeros((1, 128), jnp.float32).at[0, 0:N_AGG].set(b_agg)

    dz, sums, maxtab = pl.pallas_call(
        _pass_a,
        grid=(NB,),
        in_specs=[
            pl.BlockSpec((B, IN_DIM), lambda i: (i, 0)),
            pl.BlockSpec((1, 1, B), lambda i: (i, 0, 0)),
            pl.BlockSpec((1, B, 1), lambda i: (i, 0, 0)),
            pl.BlockSpec((IN_DIM, FEAT_DIM), lambda i: (0, 0)),
            pl.BlockSpec((1, FEAT_DIM), lambda i: (0, 0)),
            pl.BlockSpec((FEAT_DIM, 128), lambda i: (0, 0)),
            pl.BlockSpec((1, 128), lambda i: (0, 0)),
            pl.BlockSpec((FEAT_DIM, N_AGG * OUT_DIM), lambda i: (0, 0)),
            pl.BlockSpec((IN_DIM + FEAT_DIM, OUT_DIM), lambda i: (0, 0)),
        ],
        out_specs=[
            pl.BlockSpec((B, 128), lambda i: (i, 0)),
            pl.BlockSpec((S, 128), lambda i: (0, 0)),
            pl.BlockSpec((S, N_AGG * FEAT_DIM), lambda i: (0, 0)),
        ],
        out_shape=[
            jax.ShapeDtypeStruct((N, 128), jnp.float32),
            jax.ShapeDtypeStruct((S, 128), jnp.float32),
            jax.ShapeDtypeStruct((S, N_AGG * FEAT_DIM), jnp.float32),
        ],
        scratch_shapes=[
            pltpu.VMEM((S, 128), jnp.float32),
            pltpu.VMEM((S, N_AGG * FEAT_DIM), jnp.float32),
            pltpu.VMEM((B, N_AGG * FEAT_DIM), jnp.float32),
        ],
        compiler_params=pltpu.CompilerParams(
            dimension_semantics=("arbitrary",)),
    )(x, brow, bcol, W_in, b_in.reshape(1, -1), WaggP, baggP, Wm_cat, Wd)

    h = pl.pallas_call(
        _pass_b,
        grid=(NB,),
        in_specs=[
            pl.BlockSpec((B, 128), lambda i: (i, 0)),
            pl.BlockSpec((1, B, 1), lambda i: (i, 0, 0)),
            pl.BlockSpec((S, 128), lambda i: (0, 0)),
            pl.BlockSpec((S, N_AGG * FEAT_DIM), lambda i: (0, 0)),
            pl.BlockSpec((N_AGG * FEAT_DIM, OUT_DIM), lambda i: (0, 0)),
            pl.BlockSpec((1, OUT_DIM), lambda i: (0, 0)),
        ],
        out_specs=pl.BlockSpec((B, OUT_DIM), lambda i: (i, 0)),
        out_shape=jax.ShapeDtypeStruct((N, OUT_DIM), jnp.float32),
        scratch_shapes=[pltpu.VMEM((S, OUT_DIM), jnp.float32)],
        compiler_params=pltpu.CompilerParams(
            dimension_semantics=("arbitrary",)),
    )(dz, bcol, sums, maxtab, Wx_cat, b_out.reshape(1, -1))
    return h
